# Initial kernel scaffold; baseline (speedup 1.0000x reference)
import jax
import jax.numpy as jnp
from jax.experimental import pallas as pl


def kernel(points, cluster_ids, spatial_features, spatial_features_2d, common_cluster_ids, spatial_features_stride):
    raise NotImplementedError("write your pallas kernel here")


# speedup vs baseline: 6.2704x; 6.2704x over previous
import functools

import jax
import jax.numpy as jnp
from jax import lax
from jax.experimental import pallas as pl
from jax.experimental.pallas import tpu as pltpu
from jax.experimental.pallas import tpu_sc as plsc

_PC_RANGE = (-75.2, -75.2, -2.0, 75.2, 75.2, 4.0)
_VOX = (0.1, 0.1, 0.15)
_B = 2
_N = 16384
_K = 256
_H = 188
_W = 188
_HW = _H * _W
_C1 = 256
_CG = 2 * _C1
_S = 10


def _fps_body(xyz_ref, idx_ref, kpx_ref, kpy_ref, dist_ref):
    row = lax.broadcasted_iota(jnp.int32, (128, 128), 0)
    col = lax.broadcasted_iota(jnp.int32, (128, 128), 1)
    flat = row * 128 + col
    big = jnp.int32(1 << 30)
    zero = jnp.zeros((128, 128), jnp.float32)

    dist_ref[...] = jnp.full((_B, 128, 128), 1e10, jnp.float32)

    def pick(b, idx):
        eq = flat == idx
        px = jnp.sum(jnp.where(eq, xyz_ref[b, 0, :, :], zero))
        py = jnp.sum(jnp.where(eq, xyz_ref[b, 1, :, :], zero))
        pz = jnp.sum(jnp.where(eq, xyz_ref[b, 2, :, :], zero))
        return px, py, pz

    carries = []
    for b in range(_B):
        idx_ref[b, 0] = jnp.int32(0)
        px, py, pz = pick(b, jnp.int32(0))
        kpx_ref[b, 0] = px
        kpy_ref[b, 0] = py
        carries += [px, py, pz]

    def step(i, c):
        out = []
        for b in range(_B):
            px, py, pz = c[3 * b], c[3 * b + 1], c[3 * b + 2]
            dx = xyz_ref[b, 0, :, :] - px
            dy = xyz_ref[b, 1, :, :] - py
            dz = xyz_ref[b, 2, :, :] - pz
            d = (dx * dx + dy * dy) + dz * dz
            dist = jnp.minimum(dist_ref[b, :, :], d)
            dist_ref[b, :, :] = dist
            m = jnp.max(dist)
            nxt = jnp.min(jnp.where(dist == m, flat, big))
            idx_ref[b, i] = nxt
            npx, npy, npz = pick(b, nxt)
            kpx_ref[b, i] = npx
            kpy_ref[b, i] = npy
            out += [npx, npy, npz]
        return tuple(out)

    lax.fori_loop(1, _K, step, tuple(carries))


def _run_fps(xyz):
    return pl.pallas_call(
        _fps_body,
        out_shape=(
            jax.ShapeDtypeStruct((_B, _K), jnp.int32),
            jax.ShapeDtypeStruct((_B, _K), jnp.float32),
            jax.ShapeDtypeStruct((_B, _K), jnp.float32),
        ),
        out_specs=(
            pl.BlockSpec(memory_space=pltpu.SMEM),
            pl.BlockSpec(memory_space=pltpu.SMEM),
            pl.BlockSpec(memory_space=pltpu.SMEM),
        ),
        scratch_shapes=[pltpu.VMEM((_B, 128, 128), jnp.float32)],
    )(xyz)


def _sc_interp(sf_flat, sf2_flat, idxfull, w, mbias):
    mesh = plsc.VectorSubcoreMesh(core_axis_name="c", subcore_axis_name="s")

    @functools.partial(
        pl.kernel,
        mesh=mesh,
        compiler_params=pltpu.CompilerParams(needs_layout_passes=False),
        out_type=jax.ShapeDtypeStruct((_B * _CG, 16), jnp.float32),
        scratch_types=[
            pltpu.VMEM((8, 128), jnp.int32),
            pltpu.VMEM((4 * _K,), jnp.float32),
            pltpu.VMEM((_B, 4 * _K), jnp.float32),
            pltpu.VMEM((_B, _S, _K), jnp.float32),
            pltpu.VMEM((256,), jnp.float32),
            pltpu.VMEM((16,), jnp.float32),
            pltpu.SemaphoreType.DMA,
        ],
    )
    def k(sf_ref, sf2_ref, idx_hbm, w_hbm, m_hbm, out_hbm,
          idx_v, vals_v, w_v, m_v, mat_v, out_v, sem):
        cid = lax.axis_index("c")
        sid = lax.axis_index("s")
        pltpu.sync_copy(w_hbm, w_v)
        pltpu.sync_copy(m_hbm, m_v)
        iota16 = lax.broadcasted_iota(jnp.int32, (16,), 0)

        for core_val, table in ((0, sf_ref), (1, sf2_ref)):
            @pl.when(cid == core_val)
            def _():
                wid = core_val * 16 + sid

                def task(q, carry):
                    p = wid * 32 + q
                    b = p % 2
                    cg = p // 2
                    pltpu.sync_copy(idx_hbm.at[b, cg], idx_v)
                    cps = []
                    for j in range(8):
                        cps.append(pltpu.async_copy(
                            table.at[idx_v.at[j]],
                            vals_v.at[pl.ds(j * 128, 128)], sem))
                    for cp in cps:
                        cp.wait()
                    maccs = [jnp.full((16,), -3e38, jnp.float32)
                             for _ in range(_S)]
                    for t in range(16):
                        sl = pl.ds(t * 16, 16)
                        va = vals_v[pl.ds(0 * _K + t * 16, 16)]
                        vb = vals_v[pl.ds(1 * _K + t * 16, 16)]
                        vc = vals_v[pl.ds(2 * _K + t * 16, 16)]
                        vd = vals_v[pl.ds(3 * _K + t * 16, 16)]
                        wa = w_v[b, pl.ds(0 * _K + t * 16, 16)]
                        wb = w_v[b, pl.ds(1 * _K + t * 16, 16)]
                        wc = w_v[b, pl.ds(2 * _K + t * 16, 16)]
                        wd = w_v[b, pl.ds(3 * _K + t * 16, 16)]
                        contrib = ((va * wa + vb * wb) + vc * wc) + vd * wd
                        for s in range(_S):
                            maccs[s] = jnp.maximum(
                                maccs[s], contrib + m_v[b, s, sl])
                    for s in range(_S):
                        mat_v[pl.ds(s * 16, 16)] = maccs[s]
                    acc = jnp.full((16,), -3e38, jnp.float32)
                    row16 = iota16 * 16
                    for l in range(16):
                        g = plsc.load_gather(mat_v, [row16 + l])
                        acc = jnp.maximum(acc, g)
                    out_v[...] = acc
                    pltpu.sync_copy(out_v, out_hbm.at[p])
                    return carry

                lax.fori_loop(0, 32, task, jnp.int32(0))

    return k(sf_flat, sf2_flat, idxfull, w, mbias)


def kernel(points, cluster_ids, spatial_features, spatial_features_2d,
           common_cluster_ids, spatial_features_stride):
    xyz = jnp.transpose(points[:, :, :3], (0, 2, 1)).reshape(_B, 3, 128, 128)
    idxs, kpx, kpy = _run_fps(xyz)

    lbl = jnp.take_along_axis(cluster_ids, idxs, axis=1)
    x = (kpx - _PC_RANGE[0]) / _VOX[0] / spatial_features_stride
    y = (kpy - _PC_RANGE[1]) / _VOX[1] / spatial_features_stride
    x0 = jnp.floor(x).astype(jnp.int32)
    x1 = x0 + 1
    y0 = jnp.floor(y).astype(jnp.int32)
    y1 = y0 + 1
    x0 = jnp.clip(x0, 0, _W - 1)
    x1 = jnp.clip(x1, 0, _W - 1)
    y0 = jnp.clip(y0, 0, _H - 1)
    y1 = jnp.clip(y1, 0, _H - 1)
    x0f = x0.astype(jnp.float32)
    x1f = x1.astype(jnp.float32)
    y0f = y0.astype(jnp.float32)
    y1f = y1.astype(jnp.float32)
    wa = (x1f - x) * (y1f - y)
    wb = (x1f - x) * (y - y0f)
    wc = (x - x0f) * (y1f - y)
    wd = (x - x0f) * (y - y0f)
    offs = jnp.stack([y0 * _W + x0, y1 * _W + x0,
                      y0 * _W + x1, y1 * _W + x1], axis=1)
    offs = offs.reshape(_B, 4 * _K)
    w = jnp.stack([wa, wb, wc, wd], axis=1).reshape(_B, 4 * _K)
    w = w.astype(jnp.float32)
    base = (jnp.arange(_B, dtype=jnp.int32)[:, None] * _C1
            + (jnp.arange(_CG, dtype=jnp.int32) % _C1)[None, :]) * _HW
    idxfull = (base[:, :, None] + offs[:, None, :]).reshape(_B, _CG, 8, 128)
    mbias = jnp.where(lbl[:, None, :] == common_cluster_ids[:, :, None],
                      0.0, -1e10).astype(jnp.float32)

    out = _sc_interp(spatial_features.reshape(-1),
                     spatial_features_2d.reshape(-1),
                     idxfull, w, mbias)
    return out.reshape(_CG, _B, 16)[:, :, :_S].transpose(1, 2, 0)


# speedup vs baseline: 7.1795x; 1.1450x over previous
import functools

import jax
import jax.numpy as jnp
from jax import lax
from jax.experimental import pallas as pl
from jax.experimental.pallas import tpu as pltpu
from jax.experimental.pallas import tpu_sc as plsc

_PC_RANGE = (-75.2, -75.2, -2.0, 75.2, 75.2, 4.0)
_VOX = (0.1, 0.1, 0.15)
_B = 2
_N = 16384
_K = 256
_H = 188
_W = 188
_HW = _H * _W
_C1 = 256
_CG = 2 * _C1
_S = 10


def _fps_body(xyz_ref, idx_ref, kpx_ref, kpy_ref, dist_ref):
    row = lax.broadcasted_iota(jnp.int32, (128, 128), 0)
    col = lax.broadcasted_iota(jnp.int32, (128, 128), 1)
    flat = row * 128 + col
    big = jnp.int32(1 << 30)
    zero = jnp.zeros((128, 128), jnp.float32)

    dist_ref[...] = jnp.full((_B, 128, 128), 1e10, jnp.float32)

    def pick(b, idx):
        eq = flat == idx
        px = jnp.sum(jnp.where(eq, xyz_ref[b, 0, :, :], zero))
        py = jnp.sum(jnp.where(eq, xyz_ref[b, 1, :, :], zero))
        pz = jnp.sum(jnp.where(eq, xyz_ref[b, 2, :, :], zero))
        return px, py, pz

    carries = []
    for b in range(_B):
        idx_ref[b, 0] = jnp.int32(0)
        px, py, pz = pick(b, jnp.int32(0))
        kpx_ref[b, 0] = px
        kpy_ref[b, 0] = py
        carries += [px, py, pz]

    def step(i, c):
        out = []
        for b in range(_B):
            px, py, pz = c[3 * b], c[3 * b + 1], c[3 * b + 2]
            dx = xyz_ref[b, 0, :, :] - px
            dy = xyz_ref[b, 1, :, :] - py
            dz = xyz_ref[b, 2, :, :] - pz
            d = (dx * dx + dy * dy) + dz * dz
            dist = jnp.minimum(dist_ref[b, :, :], d)
            dist_ref[b, :, :] = dist
            m = jnp.max(dist)
            nxt = jnp.min(jnp.where(dist == m, flat, big))
            idx_ref[b, i] = nxt
            npx, npy, npz = pick(b, nxt)
            kpx_ref[b, i] = npx
            kpy_ref[b, i] = npy
            out += [npx, npy, npz]
        return tuple(out)

    lax.fori_loop(1, _K, step, tuple(carries))


def _run_fps(xyz):
    return pl.pallas_call(
        _fps_body,
        out_shape=(
            jax.ShapeDtypeStruct((_B, _K), jnp.int32),
            jax.ShapeDtypeStruct((_B, _K), jnp.float32),
            jax.ShapeDtypeStruct((_B, _K), jnp.float32),
        ),
        out_specs=(
            pl.BlockSpec(memory_space=pltpu.SMEM),
            pl.BlockSpec(memory_space=pltpu.SMEM),
            pl.BlockSpec(memory_space=pltpu.SMEM),
        ),
        scratch_shapes=[pltpu.VMEM((_B, 128, 128), jnp.float32)],
    )(xyz)


def _sc_interp(sf, sf2, ys, xs, w, mbias):
    mesh = plsc.VectorSubcoreMesh(core_axis_name="c", subcore_axis_name="s")

    @functools.partial(
        pl.kernel,
        mesh=mesh,
        compiler_params=pltpu.CompilerParams(needs_layout_passes=False),
        out_type=jax.ShapeDtypeStruct((_B * _CG, 16), jnp.float32),
        scratch_types=[
            pltpu.VMEM((_H, _W), jnp.float32),
            pltpu.VMEM((_H, _W), jnp.float32),
            pltpu.VMEM((_B, 4 * _K), jnp.int32),
            pltpu.VMEM((_B, 4 * _K), jnp.int32),
            pltpu.VMEM((_B, 4 * _K), jnp.float32),
            pltpu.VMEM((_B, _S, _K), jnp.float32),
            pltpu.VMEM((256,), jnp.float32),
            pltpu.VMEM((16,), jnp.float32),
            pltpu.SemaphoreType.DMA,
            pltpu.SemaphoreType.DMA,
        ],
    )
    def k(sf_ref, sf2_ref, ys_hbm, xs_hbm, w_hbm, m_hbm, out_hbm,
          plane0_v, plane1_v, ys_v, xs_v, w_v, m_v, mat_v, out_v,
          sem0, sem1):
        cid = lax.axis_index("c")
        sid = lax.axis_index("s")
        pltpu.sync_copy(ys_hbm, ys_v)
        pltpu.sync_copy(xs_hbm, xs_v)
        pltpu.sync_copy(w_hbm, w_v)
        pltpu.sync_copy(m_hbm, m_v)
        iota16 = lax.broadcasted_iota(jnp.int32, (16,), 0)
        row16 = iota16 * 16

        for core_val, table in ((0, sf_ref), (1, sf2_ref)):
            @pl.when(cid == core_val)
            def _():
                wid = core_val * 16 + sid

                def start(q, buf, sem):
                    p = wid * 32 + q
                    b = p % 2
                    cl = (p // 2) % _C1
                    pltpu.async_copy(table.at[b, cl], buf, sem)

                def wait(buf, sem):
                    pltpu.make_async_copy(table.at[0, 0], buf, sem).wait()

                def compute(q, buf):
                    p = wid * 32 + q
                    b = p % 2
                    maccs = [jnp.full((16,), -3e38, jnp.float32)
                             for _ in range(_S)]
                    for t in range(16):
                        sl = pl.ds(t * 16, 16)
                        vs = []
                        for c4 in range(4):
                            s4 = pl.ds(c4 * _K + t * 16, 16)
                            yi = ys_v[b, s4]
                            xi = xs_v[b, s4]
                            vs.append(plsc.load_gather(buf, [yi, xi]))
                        wa = w_v[b, pl.ds(0 * _K + t * 16, 16)]
                        wb = w_v[b, pl.ds(1 * _K + t * 16, 16)]
                        wc = w_v[b, pl.ds(2 * _K + t * 16, 16)]
                        wd = w_v[b, pl.ds(3 * _K + t * 16, 16)]
                        contrib = (((vs[0] * wa + vs[1] * wb)
                                    + vs[2] * wc) + vs[3] * wd)
                        for s in range(_S):
                            maccs[s] = jnp.maximum(
                                maccs[s], contrib + m_v[b, s, sl])
                    for s in range(_S):
                        mat_v[pl.ds(s * 16, 16)] = maccs[s]
                    acc = jnp.full((16,), -3e38, jnp.float32)
                    for l in range(16):
                        g = plsc.load_gather(mat_v, [row16 + l])
                        acc = jnp.maximum(acc, g)
                    out_v[...] = acc
                    pltpu.sync_copy(out_v, out_hbm.at[p])

                start(0, plane0_v, sem0)

                def pair(g, carry):
                    q0 = g * 2
                    start(q0 + 1, plane1_v, sem1)
                    wait(plane0_v, sem0)
                    compute(q0, plane0_v)

                    @pl.when(g < 15)
                    def _():
                        start(q0 + 2, plane0_v, sem0)

                    wait(plane1_v, sem1)
                    compute(q0 + 1, plane1_v)
                    return carry

                lax.fori_loop(0, 16, pair, jnp.int32(0))

    return k(sf, sf2, ys, xs, w, mbias)


def kernel(points, cluster_ids, spatial_features, spatial_features_2d,
           common_cluster_ids, spatial_features_stride):
    xyz = jnp.transpose(points[:, :, :3], (0, 2, 1)).reshape(_B, 3, 128, 128)
    idxs, kpx, kpy = _run_fps(xyz)

    lbl = jnp.take_along_axis(cluster_ids, idxs, axis=1)
    x = (kpx - _PC_RANGE[0]) / _VOX[0] / spatial_features_stride
    y = (kpy - _PC_RANGE[1]) / _VOX[1] / spatial_features_stride
    x0 = jnp.floor(x).astype(jnp.int32)
    x1 = x0 + 1
    y0 = jnp.floor(y).astype(jnp.int32)
    y1 = y0 + 1
    x0 = jnp.clip(x0, 0, _W - 1)
    x1 = jnp.clip(x1, 0, _W - 1)
    y0 = jnp.clip(y0, 0, _H - 1)
    y1 = jnp.clip(y1, 0, _H - 1)
    x0f = x0.astype(jnp.float32)
    x1f = x1.astype(jnp.float32)
    y0f = y0.astype(jnp.float32)
    y1f = y1.astype(jnp.float32)
    wa = (x1f - x) * (y1f - y)
    wb = (x1f - x) * (y - y0f)
    wc = (x - x0f) * (y1f - y)
    wd = (x - x0f) * (y - y0f)
    ys = jnp.stack([y0, y1, y0, y1], axis=1).reshape(_B, 4 * _K)
    xs = jnp.stack([x0, x0, x1, x1], axis=1).reshape(_B, 4 * _K)
    w = jnp.stack([wa, wb, wc, wd], axis=1).reshape(_B, 4 * _K)
    w = w.astype(jnp.float32)
    mbias = jnp.where(lbl[:, None, :] == common_cluster_ids[:, :, None],
                      0.0, -1e10).astype(jnp.float32)

    out = _sc_interp(spatial_features, spatial_features_2d,
                     ys, xs, w, mbias)
    return out.reshape(_CG, _B, 16)[:, :, :_S].transpose(1, 2, 0)


# speedup vs baseline: 16.1788x; 2.2535x over previous
import functools

import jax
import jax.numpy as jnp
from jax import lax
from jax.experimental import pallas as pl
from jax.experimental.pallas import tpu as pltpu
from jax.experimental.pallas import tpu_sc as plsc

_PC_RANGE = (-75.2, -75.2, -2.0, 75.2, 75.2, 4.0)
_VOX = (0.1, 0.1, 0.15)
_B = 2
_N = 16384
_K = 256
_H = 188
_W = 188
_HW = _H * _W
_C1 = 256
_CG = 2 * _C1
_S = 10


def _fps_body(xyz_ref, idx_ref, kpx_ref, kpy_ref, dist_ref):
    row = lax.broadcasted_iota(jnp.int32, (128, 128), 0)
    col = lax.broadcasted_iota(jnp.int32, (128, 128), 1)
    flat = row * 128 + col
    zero = jnp.zeros((128, 128), jnp.float32)

    dist_ref[...] = jnp.full((_B, 128, 128), 1e10, jnp.float32)

    def pick(b, idx):
        eq = flat == idx
        px = jnp.sum(jnp.where(eq, xyz_ref[b, 0, :, :], zero))
        py = jnp.sum(jnp.where(eq, xyz_ref[b, 1, :, :], zero))
        pz = jnp.sum(jnp.where(eq, xyz_ref[b, 2, :, :], zero))
        return px, py, pz

    def comb(a, b):
        pred = (a[0] > b[0]) | ((a[0] == b[0]) & (a[1] < b[1]))
        return tuple(jnp.where(pred, ai, bi) for ai, bi in zip(a, b))

    def radix_stage(t, shifts, axis):
        cands = [t] + [tuple(pltpu.roll(v, s, axis) for v in t)
                       for s in shifts]
        while len(cands) > 1:
            nxt_c = [comb(cands[j], cands[j + 1])
                     for j in range(0, len(cands) - 1, 2)]
            if len(cands) % 2:
                nxt_c.append(cands[-1])
            cands = nxt_c
        return cands[0]

    def sublane_tournament(dist, X, Y, Z):
        t = (dist, flat, X, Y, Z)
        n = 128
        while n > 8:
            h = n // 2
            t = comb(tuple(v[:h] for v in t), tuple(v[h:n] for v in t))
            n = h
        for shifts in ((2, 4, 6), (1,)):
            t = radix_stage(t, shifts, 0)
        return tuple(v[0:1, :] for v in t)

    carries = []
    for b in range(_B):
        px, py, pz = pick(b, jnp.int32(0))
        carries += [jnp.full((1, 128), px, jnp.float32),
                    jnp.full((1, 128), py, jnp.float32),
                    jnp.full((1, 128), pz, jnp.float32)]

    idx_ref[0, :, :] = jnp.zeros((2, 128), jnp.int32)
    kpx_ref[0, :, :] = jnp.concatenate([carries[0], carries[3]], 0)
    kpy_ref[0, :, :] = jnp.concatenate([carries[1], carries[4]], 0)

    def step(i, c):
        ts = []
        for b in range(_B):
            pxr, pyr, pzr = c[3 * b], c[3 * b + 1], c[3 * b + 2]
            X = xyz_ref[b, 0, :, :]
            Y = xyz_ref[b, 1, :, :]
            Z = xyz_ref[b, 2, :, :]
            dx = X - pxr
            dy = Y - pyr
            dz = Z - pzr
            d = (dx * dx + dy * dy) + dz * dz
            dist = jnp.minimum(dist_ref[b, :, :], d)
            dist_ref[b, :, :] = dist
            ts.append(sublane_tournament(dist, X, Y, Z))
        tt = tuple(jnp.concatenate([a, b], axis=0) for a, b in zip(*ts))
        for shifts in (tuple(range(8, 128, 8)), tuple(range(1, 8))):
            tt = radix_stage(tt, shifts, 1)
        idx_ref[i, :, :] = tt[1]
        kpx_ref[i, :, :] = tt[2]
        kpy_ref[i, :, :] = tt[3]
        out = []
        for b in range(_B):
            out += [tt[2][b:b + 1, :], tt[3][b:b + 1, :], tt[4][b:b + 1, :]]
        return tuple(out)

    lax.fori_loop(1, _K, step, tuple(carries))


def _run_fps(xyz):
    idx8, kpx8, kpy8 = pl.pallas_call(
        _fps_body,
        out_shape=(
            jax.ShapeDtypeStruct((_K, 2, 128), jnp.int32),
            jax.ShapeDtypeStruct((_K, 2, 128), jnp.float32),
            jax.ShapeDtypeStruct((_K, 2, 128), jnp.float32),
        ),
        scratch_shapes=[pltpu.VMEM((_B, 128, 128), jnp.float32)],
    )(xyz)
    idxs = jnp.stack([idx8[:, 0, 0], idx8[:, 1, 0]])
    kpx = jnp.stack([kpx8[:, 0, 0], kpx8[:, 1, 0]])
    kpy = jnp.stack([kpy8[:, 0, 0], kpy8[:, 1, 0]])
    return idxs, kpx, kpy


def _sc_interp(sf, sf2, ys, xs, w, mbias):
    mesh = plsc.VectorSubcoreMesh(core_axis_name="c", subcore_axis_name="s")

    @functools.partial(
        pl.kernel,
        mesh=mesh,
        compiler_params=pltpu.CompilerParams(needs_layout_passes=False),
        out_type=jax.ShapeDtypeStruct((_B * _CG, 16), jnp.float32),
        scratch_types=[
            pltpu.VMEM((_H, _W), jnp.float32),
            pltpu.VMEM((_H, _W), jnp.float32),
            pltpu.VMEM((_B, 4 * _K), jnp.int32),
            pltpu.VMEM((_B, 4 * _K), jnp.int32),
            pltpu.VMEM((_B, 4 * _K), jnp.float32),
            pltpu.VMEM((_B, _S, _K), jnp.float32),
            pltpu.VMEM((256,), jnp.float32),
            pltpu.VMEM((16,), jnp.float32),
            pltpu.SemaphoreType.DMA,
            pltpu.SemaphoreType.DMA,
        ],
    )
    def k(sf_ref, sf2_ref, ys_hbm, xs_hbm, w_hbm, m_hbm, out_hbm,
          plane0_v, plane1_v, ys_v, xs_v, w_v, m_v, mat_v, out_v,
          sem0, sem1):
        cid = lax.axis_index("c")
        sid = lax.axis_index("s")
        pltpu.sync_copy(ys_hbm, ys_v)
        pltpu.sync_copy(xs_hbm, xs_v)
        pltpu.sync_copy(w_hbm, w_v)
        pltpu.sync_copy(m_hbm, m_v)
        iota16 = lax.broadcasted_iota(jnp.int32, (16,), 0)
        row16 = iota16 * 16

        for core_val, table in ((0, sf_ref), (1, sf2_ref)):
            @pl.when(cid == core_val)
            def _():
                wid = core_val * 16 + sid

                def start(q, buf, sem):
                    p = wid * 32 + q
                    b = p % 2
                    cl = (p // 2) % _C1
                    pltpu.async_copy(table.at[b * _C1 + cl], buf, sem)

                def wait(buf, sem):
                    pltpu.make_async_copy(table.at[0], buf, sem).wait()

                def compute(q, buf):
                    p = wid * 32 + q
                    b = p % 2
                    maccs = [jnp.full((16,), -3e38, jnp.float32)
                             for _ in range(_S)]
                    for t in range(16):
                        sl = pl.ds(t * 16, 16)
                        vs = []
                        for c4 in range(4):
                            s4 = pl.ds(c4 * _K + t * 16, 16)
                            yi = ys_v[b, s4]
                            xi = xs_v[b, s4]
                            vs.append(plsc.load_gather(buf, [yi, xi]))
                        wa = w_v[b, pl.ds(0 * _K + t * 16, 16)]
                        wb = w_v[b, pl.ds(1 * _K + t * 16, 16)]
                        wc = w_v[b, pl.ds(2 * _K + t * 16, 16)]
                        wd = w_v[b, pl.ds(3 * _K + t * 16, 16)]
                        contrib = (((vs[0] * wa + vs[1] * wb)
                                    + vs[2] * wc) + vs[3] * wd)
                        for s in range(_S):
                            maccs[s] = jnp.maximum(
                                maccs[s], contrib + m_v[b, s, sl])
                    for s in range(_S):
                        mat_v[pl.ds(s * 16, 16)] = maccs[s]
                    acc = jnp.full((16,), -3e38, jnp.float32)
                    for l in range(16):
                        g = plsc.load_gather(mat_v, [row16 + l])
                        acc = jnp.maximum(acc, g)
                    out_v[...] = acc
                    pltpu.sync_copy(out_v, out_hbm.at[p])

                start(0, plane0_v, sem0)

                def pair(g, carry):
                    q0 = g * 2
                    start(q0 + 1, plane1_v, sem1)
                    wait(plane0_v, sem0)
                    compute(q0, plane0_v)

                    @pl.when(g < 15)
                    def _():
                        start(q0 + 2, plane0_v, sem0)

                    wait(plane1_v, sem1)
                    compute(q0 + 1, plane1_v)
                    return carry

                lax.fori_loop(0, 16, pair, jnp.int32(0))

    return k(sf, sf2, ys, xs, w, mbias)


def kernel(points, cluster_ids, spatial_features, spatial_features_2d,
           common_cluster_ids, spatial_features_stride):
    xyz = jnp.transpose(points[:, :, :3], (0, 2, 1)).reshape(_B, 3, 128, 128)
    idxs, kpx, kpy = _run_fps(xyz)

    lbl = jnp.take_along_axis(cluster_ids, idxs, axis=1)
    x = (kpx - _PC_RANGE[0]) / _VOX[0] / spatial_features_stride
    y = (kpy - _PC_RANGE[1]) / _VOX[1] / spatial_features_stride
    x0 = jnp.floor(x).astype(jnp.int32)
    x1 = x0 + 1
    y0 = jnp.floor(y).astype(jnp.int32)
    y1 = y0 + 1
    x0 = jnp.clip(x0, 0, _W - 1)
    x1 = jnp.clip(x1, 0, _W - 1)
    y0 = jnp.clip(y0, 0, _H - 1)
    y1 = jnp.clip(y1, 0, _H - 1)
    x0f = x0.astype(jnp.float32)
    x1f = x1.astype(jnp.float32)
    y0f = y0.astype(jnp.float32)
    y1f = y1.astype(jnp.float32)
    wa = (x1f - x) * (y1f - y)
    wb = (x1f - x) * (y - y0f)
    wc = (x - x0f) * (y1f - y)
    wd = (x - x0f) * (y - y0f)
    ys = jnp.stack([y0, y1, y0, y1], axis=1).reshape(_B, 4 * _K)
    xs = jnp.stack([x0, x0, x1, x1], axis=1).reshape(_B, 4 * _K)
    w = jnp.stack([wa, wb, wc, wd], axis=1).reshape(_B, 4 * _K)
    w = w.astype(jnp.float32)
    mbias = jnp.where(lbl[:, None, :] == common_cluster_ids[:, :, None],
                      0.0, -1e10).astype(jnp.float32)

    out = _sc_interp(spatial_features.reshape(_B * _C1, _H, _W),
                     spatial_features_2d.reshape(_B * _C1, _H, _W),
                     ys, xs, w, mbias)
    return out.reshape(_CG, _B, 16)[:, :, :_S].transpose(1, 2, 0)
